# baseline (speedup 1.0000x reference)
import functools

import jax
import jax.numpy as jnp
from jax.experimental import pallas as pl
from jax.experimental.pallas import tpu as pltpu

_FEATS = 32768
_ROWS = 128
_BR = 64
_EPS = 1e-5
_BASE_SPARSITY = 0.1


def _tree_sum(v):
    f = v.shape[-1]
    while f > 128:
        f //= 2
        v = v[:, :f] + v[:, f:]
    return jnp.sum(v, axis=-1, keepdims=True)


def _asl_body(vs_ref, x_ref, g_ref, b_ref, o_ref, key_ref, k_ref):
    @pl.when(pl.program_id(0) == 0)
    def _():
        avg = jnp.clip(_tree_sum(vs_ref[...])[0, 0] * (1.0 / _FEATS),
                       0.1, 2.0)
        sp = jnp.clip(_BASE_SPARSITY * (1.0 + 0.5 * (avg - 1.0)), 0.05, 0.25)
        k_ref[0] = jnp.maximum(1, (sp * _FEATS).astype(jnp.int32))

    k = k_ref[0]

    x = x_ref[...]
    inv_f = 1.0 / _FEATS
    mean = _tree_sum(x) * inv_f
    msq = _tree_sum(x * x) * inv_f
    var = msq - mean * mean
    xn = (x - mean) * jax.lax.rsqrt(var + _EPS) * g_ref[...] + b_ref[...]
    o_ref[...] = xn

    i32 = jax.lax.bitcast_convert_type(xn, jnp.int32)
    s = i32 ^ ((i32 >> 31) & jnp.int32(0x7FFFFFFF))
    key_ref[...] = s

    def bit_step(idx, t):
        b = 31 - idx
        cand = t ^ (jnp.int32(1) << b)
        cnt = _tree_sum((key_ref[...] >= cand).astype(jnp.int32))
        return jnp.where(cnt >= k, cand, t)

    t = jnp.full((x.shape[0], 1), jnp.int32(-(2 ** 31)))
    for idx in range(32):
        t = bit_step(jnp.int32(idx), t)

    o_ref[...] = jnp.where(key_ref[...] >= t, o_ref[...], 0.0)


@jax.jit
def kernel(x, variance_signal, gamma, beta):
    vs2 = variance_signal.reshape(1, _FEATS)
    g2 = gamma.reshape(1, _FEATS)
    b2 = beta.reshape(1, _FEATS)
    grid = (_ROWS // _BR,)
    return pl.pallas_call(
        _asl_body,
        grid=grid,
        in_specs=[
            pl.BlockSpec((1, _FEATS), lambda i: (0, 0)),
            pl.BlockSpec((_BR, _FEATS), lambda i: (i, 0)),
            pl.BlockSpec((1, _FEATS), lambda i: (0, 0)),
            pl.BlockSpec((1, _FEATS), lambda i: (0, 0)),
        ],
        out_specs=pl.BlockSpec((_BR, _FEATS), lambda i: (i, 0)),
        out_shape=jax.ShapeDtypeStruct((_ROWS, _FEATS), jnp.float32),
        scratch_shapes=[
            pltpu.VMEM((_BR, _FEATS), jnp.int32),
            pltpu.SMEM((1,), jnp.int32),
        ],
    )(vs2, x, g2, b2)


# speedup vs baseline: 1.3315x; 1.3315x over previous
import functools

import jax
import jax.numpy as jnp
from jax.experimental import pallas as pl
from jax.experimental.pallas import tpu as pltpu

_FEATS = 32768
_ROWS = 128
_BR = 64
_EPS = 1e-5
_BASE_SPARSITY = 0.1


def _tree_sum(v):
    f = v.shape[-1]
    while f > 128:
        f //= 2
        v = v[:, :f] + v[:, f:]
    return jnp.sum(v, axis=-1, keepdims=True)


def _asl_body(vs_ref, x_ref, g_ref, b_ref, o_ref, key_ref, k_ref):
    @pl.when(pl.program_id(0) == 0)
    def _():
        avg = jnp.clip(_tree_sum(vs_ref[...])[0, 0] * (1.0 / _FEATS),
                       0.1, 2.0)
        sp = jnp.clip(_BASE_SPARSITY * (1.0 + 0.5 * (avg - 1.0)), 0.05, 0.25)
        k_ref[0] = jnp.maximum(1, (sp * _FEATS).astype(jnp.int32))

    k = k_ref[0]

    x = x_ref[...]
    inv_f = 1.0 / _FEATS
    mean = _tree_sum(x) * inv_f
    msq = _tree_sum(x * x) * inv_f
    var = msq - mean * mean
    xn = (x - mean) * jax.lax.rsqrt(var + _EPS) * g_ref[...] + b_ref[...]
    o_ref[...] = xn

    i32 = jax.lax.bitcast_convert_type(xn, jnp.int32)
    s = i32 ^ ((i32 >> 31) & jnp.int32(0x7FFFFFFF))
    key_ref[...] = s

    def bit_step(idx, t):
        b = 31 - idx
        cand = t ^ (jnp.int32(1) << b)
        cnt = _tree_sum((key_ref[...] >= cand).astype(jnp.int32))
        return jnp.where(cnt >= k, cand, t)

    t0 = jnp.full((x.shape[0], 1), jnp.int32(-(2 ** 31)))
    t = jax.lax.fori_loop(0, 32, bit_step, t0)

    o_ref[...] = jnp.where(key_ref[...] >= t, o_ref[...], 0.0)


@jax.jit
def kernel(x, variance_signal, gamma, beta):
    vs2 = variance_signal.reshape(1, _FEATS)
    g2 = gamma.reshape(1, _FEATS)
    b2 = beta.reshape(1, _FEATS)
    grid = (_ROWS // _BR,)
    return pl.pallas_call(
        _asl_body,
        grid=grid,
        in_specs=[
            pl.BlockSpec((1, _FEATS), lambda i: (0, 0)),
            pl.BlockSpec((_BR, _FEATS), lambda i: (i, 0)),
            pl.BlockSpec((1, _FEATS), lambda i: (0, 0)),
            pl.BlockSpec((1, _FEATS), lambda i: (0, 0)),
        ],
        out_specs=pl.BlockSpec((_BR, _FEATS), lambda i: (i, 0)),
        out_shape=jax.ShapeDtypeStruct((_ROWS, _FEATS), jnp.float32),
        scratch_shapes=[
            pltpu.VMEM((_BR, _FEATS), jnp.int32),
            pltpu.SMEM((1,), jnp.int32),
        ],
    )(vs2, x, g2, b2)


# speedup vs baseline: 1.4843x; 1.1148x over previous
import functools

import jax
import jax.numpy as jnp
from jax.experimental import pallas as pl
from jax.experimental.pallas import tpu as pltpu

_FEATS = 32768
_ROWS = 128
_BR = 64
_EPS = 1e-5
_BASE_SPARSITY = 0.1


def _tree_sum(v):
    f = v.shape[-1]
    while f > 128:
        f //= 2
        v = v[:, :f] + v[:, f:]
    return jnp.sum(v, axis=-1, keepdims=True)


def _asl_body(vs_ref, x_ref, g_ref, b_ref, o_ref, key_ref, k_ref):
    @pl.when(pl.program_id(0) == 0)
    def _():
        avg = jnp.clip(_tree_sum(vs_ref[...])[0, 0] * (1.0 / _FEATS),
                       0.1, 2.0)
        sp = jnp.clip(_BASE_SPARSITY * (1.0 + 0.5 * (avg - 1.0)), 0.05, 0.25)
        k_ref[0] = jnp.maximum(1, (sp * _FEATS).astype(jnp.int32))

    k = k_ref[0]

    x = x_ref[...]
    inv_f = 1.0 / _FEATS
    mean = _tree_sum(x) * inv_f
    msq = _tree_sum(x * x) * inv_f
    var = msq - mean * mean
    xn = (x - mean) * jax.lax.rsqrt(var + _EPS) * g_ref[...] + b_ref[...]
    o_ref[...] = xn

    i32 = jax.lax.bitcast_convert_type(xn, jnp.int32)
    s = i32 ^ ((i32 >> 31) & jnp.int32(0x7FFFFFFF))
    key_ref[...] = s

    nacc = 4
    chunks = _FEATS // 128

    def bit_step(idx, t):
        b = 31 - idx
        cand = t ^ (jnp.int32(1) << b)
        accs = [jnp.zeros((x.shape[0], 128), jnp.int32) for _ in range(nacc)]
        for c in range(chunks):
            blk = key_ref[:, c * 128:(c + 1) * 128]
            accs[c % nacc] = accs[c % nacc] + (blk >= cand).astype(jnp.int32)
        acc = (accs[0] + accs[1]) + (accs[2] + accs[3])
        cnt = jnp.sum(acc, axis=-1, keepdims=True)
        return jnp.where(cnt >= k, cand, t)

    t0 = jnp.full((x.shape[0], 1), jnp.int32(-(2 ** 31)))
    t = jax.lax.fori_loop(0, 32, bit_step, t0)

    o_ref[...] = jnp.where(key_ref[...] >= t, o_ref[...], 0.0)


@jax.jit
def kernel(x, variance_signal, gamma, beta):
    vs2 = variance_signal.reshape(1, _FEATS)
    g2 = gamma.reshape(1, _FEATS)
    b2 = beta.reshape(1, _FEATS)
    grid = (_ROWS // _BR,)
    return pl.pallas_call(
        _asl_body,
        grid=grid,
        in_specs=[
            pl.BlockSpec((1, _FEATS), lambda i: (0, 0)),
            pl.BlockSpec((_BR, _FEATS), lambda i: (i, 0)),
            pl.BlockSpec((1, _FEATS), lambda i: (0, 0)),
            pl.BlockSpec((1, _FEATS), lambda i: (0, 0)),
        ],
        out_specs=pl.BlockSpec((_BR, _FEATS), lambda i: (i, 0)),
        out_shape=jax.ShapeDtypeStruct((_ROWS, _FEATS), jnp.float32),
        scratch_shapes=[
            pltpu.VMEM((_BR, _FEATS), jnp.int32),
            pltpu.SMEM((1,), jnp.int32),
        ],
    )(vs2, x, g2, b2)
